# baseline (speedup 1.0000x reference)
import functools

import jax
import jax.numpy as jnp
from jax import lax
from jax.experimental import pallas as pl
from jax.experimental.pallas import tpu as pltpu
from jax.experimental.pallas import tpu_sc as plsc

N = 10000
E = 320000
D = 128
R = 8

NC = 2
NS = 16
NW = NC * NS
EP = E // NW
CK = 40
UNPACK_OFFS = tuple(range(0, CK - 15, 16)) + ((CK - 16,) if CK % 16 else ())
NCHUNK = EP // CK
NBUF = 8
NBLK = NCHUNK // NBUF
NREM = NCHUNK - NBLK * NBUF
NP = 10112
RPS = NP // NS
ZR = 8

BN = 2000
NB = N // BN


def _gidx_body(et_ref, src_ref, dst_ref, o_ref):
    gidx = et_ref[...] * N + src_ref[...]
    o_ref[...] = jnp.bitwise_or(jnp.left_shift(gidx, 14), dst_ref[...])


def _gidx(edge_type, src, dst):
    et2 = edge_type.reshape(E // D, D)
    src2 = src.reshape(E // D, D)
    dst2 = dst.reshape(E // D, D)
    out = pl.pallas_call(
        _gidx_body,
        out_shape=jax.ShapeDtypeStruct((E // D, D), jnp.int32),
    )(et2, src2, dst2)
    return out.reshape(NW, NCHUNK, 1, CK)


def _xw_body(h_ref, w_ref, o_ref):
    h = h_ref[...].astype(jnp.bfloat16)
    for r in range(R):
        o_ref[r] = jnp.dot(h, w_ref[r].astype(jnp.bfloat16),
                           preferred_element_type=jnp.float32)


def _xw(h, W):
    return pl.pallas_call(
        _xw_body,
        grid=(NB,),
        in_specs=[
            pl.BlockSpec((BN, D), lambda i: (i, 0)),
            pl.BlockSpec((R, D, D), lambda i: (0, 0, 0)),
        ],
        out_specs=pl.BlockSpec((R, BN, D), lambda i: (0, i, 0)),
        out_shape=jax.ShapeDtypeStruct((R, N, D), jnp.float32),
    )(h, W)


def _combine_body(p_ref, h_ref, wr_ref, b_ref, wn_ref, oh_ref, oxw_ref):
    agg = (p_ref[0] + p_ref[1] + b_ref[...]
           + jnp.dot(h_ref[...].astype(jnp.bfloat16),
                     wr_ref[...].astype(jnp.bfloat16),
                     preferred_element_type=jnp.float32))
    hb = jnp.maximum(agg, 0.0)
    oh_ref[...] = hb
    hb16 = hb.astype(jnp.bfloat16)
    for r in range(R):
        oxw_ref[r] = jnp.dot(hb16, wn_ref[r].astype(jnp.bfloat16),
                             preferred_element_type=jnp.float32)


def _combine(p, h, Wr, b, Wn):
    return pl.pallas_call(
        _combine_body,
        grid=(NB,),
        in_specs=[
            pl.BlockSpec((2, BN, D), lambda i: (0, i, 0)),
            pl.BlockSpec((BN, D), lambda i: (i, 0)),
            pl.BlockSpec((D, D), lambda i: (0, 0)),
            pl.BlockSpec((1, D), lambda i: (0, 0)),
            pl.BlockSpec((R, D, D), lambda i: (0, 0, 0)),
        ],
        out_specs=[
            pl.BlockSpec((BN, D), lambda i: (i, 0)),
            pl.BlockSpec((R, BN, D), lambda i: (0, i, 0)),
        ],
        out_shape=[
            jax.ShapeDtypeStruct((N, D), jnp.float32),
            jax.ShapeDtypeStruct((R, N, D), jnp.float32),
        ],
    )(p, h, Wr, b.reshape(1, D), Wn)


def _final_body(p_ref, h_ref, wr_ref, b_ref, o_ref):
    o_ref[...] = (p_ref[0] + p_ref[1] + b_ref[...]
                  + jnp.dot(h_ref[...].astype(jnp.bfloat16),
                            wr_ref[...].astype(jnp.bfloat16),
                            preferred_element_type=jnp.float32))


def _final(p, h, Wr, b):
    return pl.pallas_call(
        _final_body,
        grid=(NB,),
        in_specs=[
            pl.BlockSpec((2, BN, D), lambda i: (0, i, 0)),
            pl.BlockSpec((BN, D), lambda i: (i, 0)),
            pl.BlockSpec((D, D), lambda i: (0, 0)),
            pl.BlockSpec((1, D), lambda i: (0, 0)),
        ],
        out_specs=pl.BlockSpec((BN, D), lambda i: (i, 0)),
        out_shape=jax.ShapeDtypeStruct((N, D), jnp.float32),
    )(p, h, Wr, b.reshape(1, D))


def _sc_body(xw_hbm, cmb_hbm, out_hbm,
             cmbv, idxv, dstv, rows, zbuf, acc,
             cs0, cs1, cs2, cs3, cs4, cs5, cs6, cs7, gs0, gs1, gs2, gs3, gs4, gs5, gs6, gs7, ss0, ss1, ss2, ss3, ss4, ss5, ss6, ss7):
    c = lax.axis_index("c")
    s = lax.axis_index("s")
    wid = c * NS + s
    csems = (cs0, cs1, cs2, cs3, cs4, cs5, cs6, cs7)
    gsems = (gs0, gs1, gs2, gs3, gs4, gs5, gs6, gs7)
    ssems = (ss0, ss1, ss2, ss3, ss4, ss5, ss6, ss7)

    zero16 = jnp.zeros((16,), jnp.float32)

    def zrow(i, carry):
        for j in range(D // 16):
            zbuf[i, pl.ds(j * 16, 16)] = zero16
        return carry

    lax.fori_loop(0, ZR, zrow, 0)

    def zcopy(k, carry):
        pltpu.sync_copy(zbuf, acc.at[pl.ds(s * RPS + k * ZR, ZR)])
        return carry

    lax.fori_loop(0, RPS // ZR, zcopy, 0)
    plsc.subcore_barrier()

    def c_start(i, b):
        pltpu.async_copy(cmb_hbm.at[wid, i], cmbv.at[b], csems[b])

    def c_wait(i, b):
        pltpu.make_async_copy(cmb_hbm.at[wid, i], cmbv.at[b], csems[b]).wait()

    def unpack(b):
        for off in UNPACK_OFFS:
            v = cmbv[b, 0, pl.ds(off, 16)]
            idxv[b, pl.ds(off, 16)] = lax.shift_right_logical(v, 14)
            dstv[b, pl.ds(off, 16)] = jnp.bitwise_and(v, (1 << 14) - 1)

    H = CK // 2

    def g_start(i, b):
        pltpu.async_copy(xw_hbm.at[idxv.at[b, pl.ds(0, H)]],
                         rows.at[b, pl.ds(0, H)], gsems[b])
        pltpu.async_copy(xw_hbm.at[idxv.at[b, pl.ds(H, H)]],
                         rows.at[b, pl.ds(H, H)], gsems[b])

    def g_wait(i, b):
        pltpu.make_async_copy(xw_hbm.at[idxv.at[b, pl.ds(0, H)]],
                              rows.at[b, pl.ds(0, H)], gsems[b]).wait()
        pltpu.make_async_copy(xw_hbm.at[idxv.at[b, pl.ds(H, H)]],
                              rows.at[b, pl.ds(H, H)], gsems[b]).wait()

    def s_start(i, b):
        pltpu.async_copy(rows.at[b], acc.at[dstv.at[b]], ssems[b], add=True)

    def s_wait(i, b):
        pltpu.make_async_copy(rows.at[b], acc.at[dstv.at[b]], ssems[b]).wait()

    for b in range(NBUF):
        c_start(b, b)
    for b in range(NBUF):
        c_wait(b, b)
        unpack(b)
        g_start(b, b)

    def block(k, carry):
        i0 = k * NBUF
        for b in range(NBUF):
            g_wait(i0 + b, b)
            s_start(i0 + b, b)
            c_start(i0 + NBUF + b, b)
        for b in range(NBUF):
            s_wait(i0 + b, b)
            c_wait(i0 + NBUF + b, b)
            unpack(b)
            g_start(i0 + NBUF + b, b)
        return carry

    lax.fori_loop(0, NBLK - 1, block, 0)
    i0 = (NBLK - 1) * NBUF
    for b in range(NBUF):
        g_wait(i0 + b, b)
        s_start(i0 + b, b)
        if b < NREM:
            c_start(i0 + NBUF + b, b)
    for b in range(NBUF):
        s_wait(i0 + b, b)
        if b < NREM:
            c_wait(i0 + NBUF + b, b)
            unpack(b)
            g_start(i0 + NBUF + b, b)
    for b in range(NREM):
        g_wait(i0 + NBUF + b, b)
        s_start(i0 + NBUF + b, b)
    for b in range(NREM):
        s_wait(i0 + NBUF + b, b)
    plsc.subcore_barrier()

    pltpu.sync_copy(acc.at[pl.ds(s * RPS, RPS)],
                    out_hbm.at[c, pl.ds(s * RPS, RPS)])


@functools.partial(jax.jit, static_argnames=())
def _sc_aggregate(xw_flat, cmb):
    mesh = plsc.VectorSubcoreMesh(core_axis_name="c", subcore_axis_name="s",
                                  num_cores=NC, num_subcores=NS)
    p = pl.kernel(
        _sc_body,
        out_type=jax.ShapeDtypeStruct((NC, NP, D), jnp.float32),
        mesh=mesh,
        scratch_types=[
            pltpu.VMEM((NBUF, 1, CK), jnp.int32),
            pltpu.VMEM((NBUF, CK), jnp.int32),
            pltpu.VMEM((NBUF, CK), jnp.int32),
            pltpu.VMEM((NBUF, CK, D), jnp.float32),
            pltpu.VMEM((ZR, D), jnp.float32),
            pltpu.VMEM_SHARED((NP, D), jnp.float32),
        ] + [pltpu.SemaphoreType.DMA] * (3 * NBUF),
    )(xw_flat, cmb)
    return p


def kernel(x, edge_index, edge_type, W1, Wr1, b1, W2, Wr2, b2, W3, Wr3, b3):
    assert x.shape == (N, D) and edge_type.shape == (E,)
    src = edge_index[0]
    dst = edge_index[1]
    cmb = _gidx(edge_type, src, dst)

    xw = _xw(x, W1)
    p = _sc_aggregate(xw.reshape(R * N, D), cmb)
    h, xw = _combine(p, x, Wr1, b1, W2)

    p = _sc_aggregate(xw.reshape(R * N, D), cmb)
    h, xw = _combine(p, h, Wr2, b2, W3)

    p = _sc_aggregate(xw.reshape(R * N, D), cmb)
    return _final(p, h, Wr3, b3)


# speedup vs baseline: 1.0135x; 1.0135x over previous
import functools

import jax
import jax.numpy as jnp
from jax import lax
from jax.experimental import pallas as pl
from jax.experimental.pallas import tpu as pltpu
from jax.experimental.pallas import tpu_sc as plsc

N = 10000
E = 320000
D = 128
R = 8

NC = 2
NS = 16
NW = NC * NS
EP = E // NW
CK = 40
UNPACK_OFFS = tuple(range(0, CK - 15, 16)) + ((CK - 16,) if CK % 16 else ())
NCHUNK = EP // CK
NBUF = 8
NBLK = NCHUNK // NBUF
NREM = NCHUNK - NBLK * NBUF
NP = 10112
RPS = NP // NS
ZR = 8

BN = 2000
NB = N // BN


def _gidx_body(et_ref, src_ref, dst_ref, o_ref):
    gidx = et_ref[...] * N + src_ref[...]
    o_ref[...] = jnp.bitwise_or(jnp.left_shift(gidx, 14), dst_ref[...])


def _gidx(edge_type, src, dst):
    et2 = edge_type.reshape(E // D, D)
    src2 = src.reshape(E // D, D)
    dst2 = dst.reshape(E // D, D)
    out = pl.pallas_call(
        _gidx_body,
        out_shape=jax.ShapeDtypeStruct((E // D, D), jnp.int32),
    )(et2, src2, dst2)
    return out.reshape(NW, NCHUNK, 1, CK)


def _xw_body(h_ref, w_ref, o_ref):
    h = h_ref[...]
    for r in range(R):
        o_ref[r] = jnp.dot(h, w_ref[r], preferred_element_type=jnp.float32)


def _xw(h, W):
    return pl.pallas_call(
        _xw_body,
        grid=(NB,),
        in_specs=[
            pl.BlockSpec((BN, D), lambda i: (i, 0)),
            pl.BlockSpec((R, D, D), lambda i: (0, 0, 0)),
        ],
        out_specs=pl.BlockSpec((R, BN, D), lambda i: (0, i, 0)),
        out_shape=jax.ShapeDtypeStruct((R, N, D), jnp.float32),
    )(h, W)


def _combine_body(p_ref, h_ref, wr_ref, b_ref, wn_ref, oh_ref, oxw_ref):
    agg = (p_ref[0] + p_ref[1] + b_ref[...]
           + jnp.dot(h_ref[...], wr_ref[...], preferred_element_type=jnp.float32))
    hb = jnp.maximum(agg, 0.0)
    oh_ref[...] = hb
    for r in range(R):
        oxw_ref[r] = jnp.dot(hb, wn_ref[r], preferred_element_type=jnp.float32)


def _combine(p, h, Wr, b, Wn):
    return pl.pallas_call(
        _combine_body,
        grid=(NB,),
        in_specs=[
            pl.BlockSpec((2, BN, D), lambda i: (0, i, 0)),
            pl.BlockSpec((BN, D), lambda i: (i, 0)),
            pl.BlockSpec((D, D), lambda i: (0, 0)),
            pl.BlockSpec((1, D), lambda i: (0, 0)),
            pl.BlockSpec((R, D, D), lambda i: (0, 0, 0)),
        ],
        out_specs=[
            pl.BlockSpec((BN, D), lambda i: (i, 0)),
            pl.BlockSpec((R, BN, D), lambda i: (0, i, 0)),
        ],
        out_shape=[
            jax.ShapeDtypeStruct((N, D), jnp.float32),
            jax.ShapeDtypeStruct((R, N, D), jnp.float32),
        ],
    )(p, h, Wr, b.reshape(1, D), Wn)


def _final_body(p_ref, h_ref, wr_ref, b_ref, o_ref):
    o_ref[...] = (p_ref[0] + p_ref[1] + b_ref[...]
                  + jnp.dot(h_ref[...], wr_ref[...],
                            preferred_element_type=jnp.float32))


def _final(p, h, Wr, b):
    return pl.pallas_call(
        _final_body,
        grid=(NB,),
        in_specs=[
            pl.BlockSpec((2, BN, D), lambda i: (0, i, 0)),
            pl.BlockSpec((BN, D), lambda i: (i, 0)),
            pl.BlockSpec((D, D), lambda i: (0, 0)),
            pl.BlockSpec((1, D), lambda i: (0, 0)),
        ],
        out_specs=pl.BlockSpec((BN, D), lambda i: (i, 0)),
        out_shape=jax.ShapeDtypeStruct((N, D), jnp.float32),
    )(p, h, Wr, b.reshape(1, D))


def _sc_body(xw_hbm, cmb_hbm, out_hbm,
             cmbv, idxv, dstv, rows, zbuf, acc,
             cs0, cs1, cs2, cs3, cs4, cs5, cs6, cs7, gs0, gs1, gs2, gs3, gs4, gs5, gs6, gs7, ss0, ss1, ss2, ss3, ss4, ss5, ss6, ss7):
    c = lax.axis_index("c")
    s = lax.axis_index("s")
    wid = c * NS + s
    csems = (cs0, cs1, cs2, cs3, cs4, cs5, cs6, cs7)
    gsems = (gs0, gs1, gs2, gs3, gs4, gs5, gs6, gs7)
    ssems = (ss0, ss1, ss2, ss3, ss4, ss5, ss6, ss7)

    zero16 = jnp.zeros((16,), jnp.float32)

    def zrow(i, carry):
        for j in range(D // 16):
            zbuf[i, pl.ds(j * 16, 16)] = zero16
        return carry

    lax.fori_loop(0, ZR, zrow, 0)

    def zcopy(k, carry):
        pltpu.sync_copy(zbuf, acc.at[pl.ds(s * RPS + k * ZR, ZR)])
        return carry

    lax.fori_loop(0, RPS // ZR, zcopy, 0)
    plsc.subcore_barrier()

    def c_start(i, b):
        pltpu.async_copy(cmb_hbm.at[wid, i], cmbv.at[b], csems[b])

    def c_wait(i, b):
        pltpu.make_async_copy(cmb_hbm.at[wid, i], cmbv.at[b], csems[b]).wait()

    def unpack(b):
        for off in UNPACK_OFFS:
            v = cmbv[b, 0, pl.ds(off, 16)]
            idxv[b, pl.ds(off, 16)] = lax.shift_right_logical(v, 14)
            dstv[b, pl.ds(off, 16)] = jnp.bitwise_and(v, (1 << 14) - 1)

    H = CK // 2

    def g_start(i, b):
        pltpu.async_copy(xw_hbm.at[idxv.at[b, pl.ds(0, H)]],
                         rows.at[b, pl.ds(0, H)], gsems[b])
        pltpu.async_copy(xw_hbm.at[idxv.at[b, pl.ds(H, H)]],
                         rows.at[b, pl.ds(H, H)], gsems[b])

    def g_wait(i, b):
        pltpu.make_async_copy(xw_hbm.at[idxv.at[b, pl.ds(0, H)]],
                              rows.at[b, pl.ds(0, H)], gsems[b]).wait()
        pltpu.make_async_copy(xw_hbm.at[idxv.at[b, pl.ds(H, H)]],
                              rows.at[b, pl.ds(H, H)], gsems[b]).wait()

    def s_start(i, b):
        pltpu.async_copy(rows.at[b], acc.at[dstv.at[b]], ssems[b], add=True)

    def s_wait(i, b):
        pltpu.make_async_copy(rows.at[b], acc.at[dstv.at[b]], ssems[b]).wait()

    for b in range(NBUF):
        c_start(b, b)
    for b in range(NBUF):
        c_wait(b, b)
        unpack(b)
        g_start(b, b)

    def block(k, carry):
        i0 = k * NBUF
        for b in range(NBUF):
            g_wait(i0 + b, b)
            s_start(i0 + b, b)
            c_start(i0 + NBUF + b, b)
        for b in range(NBUF):
            s_wait(i0 + b, b)
            c_wait(i0 + NBUF + b, b)
            unpack(b)
            g_start(i0 + NBUF + b, b)
        return carry

    lax.fori_loop(0, NBLK - 1, block, 0)
    i0 = (NBLK - 1) * NBUF
    for b in range(NBUF):
        g_wait(i0 + b, b)
        s_start(i0 + b, b)
        if b < NREM:
            c_start(i0 + NBUF + b, b)
    for b in range(NBUF):
        s_wait(i0 + b, b)
        if b < NREM:
            c_wait(i0 + NBUF + b, b)
            unpack(b)
            g_start(i0 + NBUF + b, b)
    for b in range(NREM):
        g_wait(i0 + NBUF + b, b)
        s_start(i0 + NBUF + b, b)
    for b in range(NREM):
        s_wait(i0 + NBUF + b, b)
    plsc.subcore_barrier()

    pltpu.sync_copy(acc.at[pl.ds(s * RPS, RPS)],
                    out_hbm.at[c, pl.ds(s * RPS, RPS)])


@functools.partial(jax.jit, static_argnames=())
def _sc_aggregate(xw_flat, cmb):
    mesh = plsc.VectorSubcoreMesh(core_axis_name="c", subcore_axis_name="s",
                                  num_cores=NC, num_subcores=NS)
    p = pl.kernel(
        _sc_body,
        out_type=jax.ShapeDtypeStruct((NC, NP, D), jnp.float32),
        mesh=mesh,
        scratch_types=[
            pltpu.VMEM((NBUF, 1, CK), jnp.int32),
            pltpu.VMEM((NBUF, CK), jnp.int32),
            pltpu.VMEM((NBUF, CK), jnp.int32),
            pltpu.VMEM((NBUF, CK, D), jnp.float32),
            pltpu.VMEM((ZR, D), jnp.float32),
            pltpu.VMEM_SHARED((NP, D), jnp.float32),
        ] + [pltpu.SemaphoreType.DMA] * (3 * NBUF),
    )(xw_flat, cmb)
    return p


def kernel(x, edge_index, edge_type, W1, Wr1, b1, W2, Wr2, b2, W3, Wr3, b3):
    assert x.shape == (N, D) and edge_type.shape == (E,)
    src = edge_index[0]
    dst = edge_index[1]
    cmb = _gidx(edge_type, src, dst)

    xw = _xw(x, W1)
    p = _sc_aggregate(xw.reshape(R * N, D), cmb)
    h, xw = _combine(p, x, Wr1, b1, W2)

    p = _sc_aggregate(xw.reshape(R * N, D), cmb)
    h, xw = _combine(p, h, Wr2, b2, W3)

    p = _sc_aggregate(xw.reshape(R * N, D), cmb)
    return _final(p, h, Wr3, b3)
